# baseline (speedup 1.0000x reference)
import functools

import jax
import jax.numpy as jnp
from jax import lax
from jax.experimental import pallas as pl
from jax.experimental.pallas import tpu as pltpu
from jax.experimental.pallas import tpu_sc as plsc

B = 4096
L = 200
D = 64
NC = 2
NS = 16
NW = NC * NS
TOTAL = B * L
PER_TILE = TOTAL // NW
IDX_MINOR = 128
CHUNK = 256
IDX_ROWS = CHUNK // IDX_MINOR
STAGES = PER_TILE // CHUNK
TILE_IDX_ROWS = PER_TILE // IDX_MINOR
NBUF = 4


def _gather_body(idx_hbm, table_hbm, out_hbm, idx_all,
                 rows0, rows1, rows2, rows3,
                 gsem0, gsem1, gsem2, gsem3,
                 ssem0, ssem1, ssem2, ssem3):
    c = lax.axis_index("c")
    s = lax.axis_index("s")
    wid = s * NC + c
    idx_base = wid * TILE_IDX_ROWS
    row_base = wid * PER_TILE

    rows = (rows0, rows1, rows2, rows3)
    gsem = (gsem0, gsem1, gsem2, gsem3)
    ssem = (ssem0, ssem1, ssem2, ssem3)

    pltpu.sync_copy(idx_hbm.at[pl.ds(idx_base, TILE_IDX_ROWS)], idx_all)

    def fire_gather(g, b):
        for j in range(IDX_ROWS):
            pltpu.async_copy(table_hbm.at[idx_all.at[g * IDX_ROWS + j]],
                             rows[b].at[pl.ds(j * IDX_MINOR, IDX_MINOR)],
                             gsem[b])

    def wait_gather(b):
        for j in range(IDX_ROWS):
            pltpu.make_async_copy(table_hbm.at[idx_all.at[0]],
                                  rows[b].at[pl.ds(j * IDX_MINOR, IDX_MINOR)],
                                  gsem[b]).wait()

    def fire_store(g, b):
        pltpu.async_copy(rows[b],
                         out_hbm.at[pl.ds(row_base + g * CHUNK, CHUNK)],
                         ssem[b])

    def wait_store(g, b):
        pltpu.make_async_copy(rows[b],
                              out_hbm.at[pl.ds(row_base + g * CHUNK, CHUNK)],
                              ssem[b]).wait()

    fire_gather(0, 0)
    fire_gather(1, 1)
    fire_gather(2, 2)
    fire_gather(3, 3)
    wait_gather(0)
    fire_store(0, 0)

    def quad(k, carry):
        g4 = 4 + 4 * k
        for b in range(NBUF):
            g = g4 + b
            wait_store(g - NBUF, b)
            fire_gather(g, b)
            pb = (b + 1) % NBUF
            wait_gather(pb)
            fire_store(g - 3, pb)
        return carry

    lax.fori_loop(0, (STAGES - NBUF) // NBUF, quad, 0)

    for t in (STAGES - 3, STAGES - 2, STAGES - 1):
        bt = t % NBUF
        wait_gather(bt)
        fire_store(t, bt)
    for t in (STAGES - 4, STAGES - 3, STAGES - 2, STAGES - 1):
        wait_store(t, t % NBUF)


def _sc_gather(idx2d, table):
    mesh = plsc.VectorSubcoreMesh(core_axis_name="c", subcore_axis_name="s",
                                  num_cores=NC, num_subcores=NS)
    fn = pl.kernel(
        _gather_body,
        out_type=jax.ShapeDtypeStruct((TOTAL, D), jnp.float32),
        mesh=mesh,
        scratch_types=[
            pltpu.VMEM((TILE_IDX_ROWS, IDX_MINOR), jnp.int32),
            pltpu.VMEM((CHUNK, D), jnp.float32),
            pltpu.VMEM((CHUNK, D), jnp.float32),
            pltpu.VMEM((CHUNK, D), jnp.float32),
            pltpu.VMEM((CHUNK, D), jnp.float32),
            pltpu.SemaphoreType.DMA,
            pltpu.SemaphoreType.DMA,
            pltpu.SemaphoreType.DMA,
            pltpu.SemaphoreType.DMA,
            pltpu.SemaphoreType.DMA,
            pltpu.SemaphoreType.DMA,
            pltpu.SemaphoreType.DMA,
            pltpu.SemaphoreType.DMA,
        ],
        compiler_params=pltpu.CompilerParams(use_tc_tiling_on_sc=False),
    )
    return fn(idx2d, table)


PB = 32
PBP = 128


def _item_body(x_ref, item_ref):
    v = x_ref[...]
    left = v[:, :D].reshape(PB, L // 2, D)
    right = v[:, D:].reshape(PB, L // 2, D)
    item_ref[...] = jnp.concatenate([left, right], axis=1)


def _item_finish(x):
    return pl.pallas_call(
        _item_body,
        grid=(B // PB,),
        in_specs=[pl.BlockSpec((PB * L // 2, 128), lambda i: (i, 0))],
        out_specs=pl.BlockSpec((PB, L, D), lambda i: (i, 0, 0)),
        out_shape=jax.ShapeDtypeStruct((B, L, D), jnp.float32),
    )(x)


def _price_body(pt_ref, w_ref, b_ref, price_ref):
    w = w_ref[...]
    bb = b_ref[...]
    for c in range(PBP):
        p_col = pt_ref[:, c][:, None]
        price_ref[c, :, :] = p_col * w + bb


def _price_emb(priceT, W, b):
    return pl.pallas_call(
        _price_body,
        grid=(B // PBP,),
        in_specs=[
            pl.BlockSpec((L, PBP), lambda i: (0, i)),
            pl.BlockSpec((1, D), lambda i: (0, 0)),
            pl.BlockSpec((1, D), lambda i: (0, 0)),
        ],
        out_specs=pl.BlockSpec((PBP, L, D), lambda i: (i, 0, 0)),
        out_shape=jax.ShapeDtypeStruct((B, L, D), jnp.float32),
    )(priceT, W, b)


@jax.jit
def kernel(item_id, price, emb_table, W, b):
    idx_perm = item_id.reshape(B, 2, L // 2).swapaxes(1, 2)
    idx2d = idx_perm.reshape(TOTAL // IDX_MINOR, IDX_MINOR)
    packed = _sc_gather(idx2d, emb_table)
    x = packed.reshape(TOTAL // 2, 128)
    item_emb = _item_finish(x)
    price_emb = _price_emb(price.T, W, b.reshape(1, D))
    return (item_emb, price_emb)


# speedup vs baseline: 1.4612x; 1.4612x over previous
import functools

import jax
import jax.numpy as jnp
from jax import lax
from jax.experimental import pallas as pl
from jax.experimental.pallas import tpu as pltpu
from jax.experimental.pallas import tpu_sc as plsc

B = 4096
L = 200
D = 64
NC = 2
NS = 16
NW = NC * NS
TOTAL = B * L
PER_TILE = TOTAL // NW
NIDX = 512
STAGES = PER_TILE // NIDX
NBUF = 2


def _gather_body(idx_hbm, table_hbm, out_hbm, idx_all, rows0, rows1,
                 gsem0, gsem1, ssem0, ssem1):
    c = lax.axis_index("c")
    s = lax.axis_index("s")
    wid = s * NC + c
    idx_base = wid * PER_TILE
    row_base = wid * PER_TILE

    rows = (rows0, rows1)
    gsem = (gsem0, gsem1)
    ssem = (ssem0, ssem1)

    pltpu.sync_copy(idx_hbm.at[pl.ds(idx_base, PER_TILE)], idx_all)

    def fire_gather(g, b):
        pltpu.async_copy(table_hbm.at[idx_all.at[pl.ds(g * NIDX, NIDX)]],
                         rows[b], gsem[b])

    def wait_gather(b):
        pltpu.make_async_copy(table_hbm.at[idx_all.at[pl.ds(0, NIDX)]],
                              rows[b], gsem[b]).wait()

    def fire_store(g, b):
        pltpu.async_copy(rows[b],
                         out_hbm.at[pl.ds(row_base + g * NIDX, NIDX)],
                         ssem[b])

    def wait_store(g, b):
        pltpu.make_async_copy(rows[b],
                              out_hbm.at[pl.ds(row_base + g * NIDX, NIDX)],
                              ssem[b]).wait()

    fire_gather(0, 0)
    fire_gather(1, 1)
    wait_gather(0)
    fire_store(0, 0)

    def pair(k, carry):
        g2 = 2 + 2 * k
        for b in range(NBUF):
            g = g2 + b
            pb = 1 - b
            wait_store(g - 2, b)
            fire_gather(g, b)
            wait_gather(pb)
            fire_store(g - 1, pb)
        return carry

    lax.fori_loop(0, (STAGES - NBUF) // NBUF, pair, 0)

    wait_gather(1)
    fire_store(STAGES - 1, 1)
    wait_store(STAGES - 2, 0)
    wait_store(STAGES - 1, 1)


def _sc_gather(idx_flat, table):
    mesh = plsc.VectorSubcoreMesh(core_axis_name="c", subcore_axis_name="s",
                                  num_cores=NC, num_subcores=NS)
    fn = pl.kernel(
        _gather_body,
        out_type=jax.ShapeDtypeStruct((TOTAL, D), jnp.float32),
        mesh=mesh,
        scratch_types=[
            pltpu.VMEM((PER_TILE,), jnp.int32),
            pltpu.VMEM((NIDX, D), jnp.float32),
            pltpu.VMEM((NIDX, D), jnp.float32),
            pltpu.SemaphoreType.DMA,
            pltpu.SemaphoreType.DMA,
            pltpu.SemaphoreType.DMA,
            pltpu.SemaphoreType.DMA,
        ],
        compiler_params=pltpu.CompilerParams(use_tc_tiling_on_sc=False),
    )
    return fn(idx_flat, table)


LBLK = 8
BBLK = 2048


def _price_body(pt_ref, w_ref, b_ref, o_ref):
    p = pt_ref[...]
    w = w_ref[...].reshape(D, 1)
    bb = b_ref[...].reshape(D, 1)
    o_ref[...] = p[:, None, :] * w[None, :, :] + bb[None, :, :]


def _price_ldb(priceT, W, b):
    return pl.pallas_call(
        _price_body,
        grid=(L // LBLK, B // BBLK),
        in_specs=[
            pl.BlockSpec((LBLK, BBLK), lambda i, j: (i, j)),
            pl.BlockSpec((1, D), lambda i, j: (0, 0)),
            pl.BlockSpec((1, D), lambda i, j: (0, 0)),
        ],
        out_specs=pl.BlockSpec((LBLK, D, BBLK), lambda i, j: (i, 0, j)),
        out_shape=jax.ShapeDtypeStruct((L, D, B), jnp.float32),
    )(priceT, W, b)


@jax.jit
def kernel(item_id, price, emb_table, W, b):
    idx_flat = item_id.T.reshape(TOTAL)
    packed = _sc_gather(idx_flat, emb_table)
    item_emb = packed.reshape(L, B, D).transpose(1, 0, 2)
    price_ldb = _price_ldb(price.T, W, b.reshape(1, D))
    price_emb = price_ldb.transpose(2, 0, 1)
    return (item_emb, price_emb)


# speedup vs baseline: 1.4655x; 1.0030x over previous
import functools

import jax
import jax.numpy as jnp
from jax import lax
from jax.experimental import pallas as pl
from jax.experimental.pallas import tpu as pltpu
from jax.experimental.pallas import tpu_sc as plsc

B = 4096
L = 200
D = 64
NC = 2
NS = 16
NW = NC * NS
TOTAL = B * L
PER_TILE = TOTAL // NW
NIDX = 512
STAGES = PER_TILE // NIDX
NBUF = 2


def _gather_body(idx_hbm, table_hbm, out_hbm, idx_all, rows0, rows1,
                 gsem0, gsem1, ssem0, ssem1):
    c = lax.axis_index("c")
    s = lax.axis_index("s")
    wid = s * NC + c
    idx_base = wid * PER_TILE
    row_base = wid * PER_TILE

    rows = (rows0, rows1)
    gsem = (gsem0, gsem1)
    ssem = (ssem0, ssem1)

    pltpu.sync_copy(idx_hbm.at[pl.ds(idx_base, PER_TILE)], idx_all)

    def fire_gather(g, b):
        pltpu.async_copy(table_hbm.at[idx_all.at[pl.ds(g * NIDX, NIDX)]],
                         rows[b], gsem[b])

    def wait_gather(b):
        pltpu.make_async_copy(table_hbm.at[idx_all.at[pl.ds(0, NIDX)]],
                              rows[b], gsem[b]).wait()

    def fire_store(g, b):
        pltpu.async_copy(rows[b],
                         out_hbm.at[pl.ds(row_base + g * NIDX, NIDX)],
                         ssem[b])

    def wait_store(g, b):
        pltpu.make_async_copy(rows[b],
                              out_hbm.at[pl.ds(row_base + g * NIDX, NIDX)],
                              ssem[b]).wait()

    fire_gather(0, 0)
    fire_gather(1, 1)
    wait_gather(0)
    fire_store(0, 0)

    def pair(k, carry):
        g2 = 2 + 2 * k
        for b in range(NBUF):
            g = g2 + b
            pb = 1 - b
            wait_store(g - 2, b)
            fire_gather(g, b)
            wait_gather(pb)
            fire_store(g - 1, pb)
        return carry

    lax.fori_loop(0, (STAGES - NBUF) // NBUF, pair, 0)

    wait_gather(1)
    fire_store(STAGES - 1, 1)
    wait_store(STAGES - 2, 0)
    wait_store(STAGES - 1, 1)


def _sc_gather(idx_flat, table):
    mesh = plsc.VectorSubcoreMesh(core_axis_name="c", subcore_axis_name="s",
                                  num_cores=NC, num_subcores=NS)
    fn = pl.kernel(
        _gather_body,
        out_type=jax.ShapeDtypeStruct((TOTAL, D), jnp.float32),
        mesh=mesh,
        scratch_types=[
            pltpu.VMEM((PER_TILE,), jnp.int32),
            pltpu.VMEM((NIDX, D), jnp.float32),
            pltpu.VMEM((NIDX, D), jnp.float32),
            pltpu.SemaphoreType.DMA,
            pltpu.SemaphoreType.DMA,
            pltpu.SemaphoreType.DMA,
            pltpu.SemaphoreType.DMA,
        ],
        compiler_params=pltpu.CompilerParams(use_tc_tiling_on_sc=False),
    )
    return fn(idx_flat, table)


LBLK = 8
BBLK = 2048


def _price_body(pt_ref, w_ref, b_ref, o_ref):
    p = pt_ref[...]
    w = w_ref[...].reshape(D, 1)
    bb = b_ref[...].reshape(D, 1)
    o_ref[...] = p[:, None, :] * w[None, :, :] + bb[None, :, :]


def _price_ldb(priceT, W, b):
    return pl.pallas_call(
        _price_body,
        grid=(L // LBLK, B // BBLK),
        in_specs=[
            pl.BlockSpec((LBLK, BBLK), lambda i, j: (i, j)),
            pl.BlockSpec((1, D), lambda i, j: (0, 0)),
            pl.BlockSpec((1, D), lambda i, j: (0, 0)),
        ],
        out_specs=pl.BlockSpec((LBLK, D, BBLK), lambda i, j: (i, 0, j)),
        out_shape=jax.ShapeDtypeStruct((L, D, B), jnp.float32),
    )(priceT, W, b)


@jax.jit
def kernel(item_id, price, emb_table, W, b):
    idx_flat = item_id.reshape(TOTAL)
    packed = _sc_gather(idx_flat, emb_table)
    item_emb = packed.reshape(B, L, D)
    price_ldb = _price_ldb(price.T, W, b.reshape(1, D))
    price_emb = price_ldb.transpose(2, 0, 1)
    return (item_emb, price_emb)


# speedup vs baseline: 1.7845x; 1.2176x over previous
import functools

import numpy as np
import jax
import jax.numpy as jnp
from jax import lax
from jax.experimental import pallas as pl
from jax.experimental.pallas import tpu as pltpu
from jax.experimental.pallas import tpu_sc as plsc

B = 4096
L = 200
D = 64
NC = 2
NS = 16
NW = NC * NS
TOTAL = B * L
PER_TILE = TOTAL // NW
NIDX = 512
STAGES = PER_TILE // NIDX
NBUF = 2


def _gather_body(idx_hbm, table_hbm, out_hbm, idx_all, rows0, rows1,
                 gsem0, gsem1, ssem0, ssem1):
    c = lax.axis_index("c")
    s = lax.axis_index("s")
    wid = s * NC + c
    base = wid * PER_TILE

    rows = (rows0, rows1)
    gsem = (gsem0, gsem1)
    ssem = (ssem0, ssem1)

    pltpu.sync_copy(idx_hbm.at[pl.ds(base, PER_TILE)], idx_all)

    def fire_gather(g, b):
        pltpu.async_copy(table_hbm.at[idx_all.at[pl.ds(g * NIDX, NIDX)]],
                         rows[b], gsem[b])

    def wait_gather(b):
        pltpu.make_async_copy(table_hbm.at[idx_all.at[pl.ds(0, NIDX)]],
                              rows[b], gsem[b]).wait()

    def fire_store(g, b):
        pltpu.async_copy(rows[b],
                         out_hbm.at[pl.ds(base + g * NIDX, NIDX)],
                         ssem[b])

    def wait_store(g, b):
        pltpu.make_async_copy(rows[b],
                              out_hbm.at[pl.ds(base + g * NIDX, NIDX)],
                              ssem[b]).wait()

    fire_gather(0, 0)
    fire_gather(1, 1)
    wait_gather(0)
    fire_store(0, 0)

    def pair(k, carry):
        g2 = 2 + 2 * k
        for b in range(NBUF):
            g = g2 + b
            pb = 1 - b
            wait_store(g - 2, b)
            fire_gather(g, b)
            wait_gather(pb)
            fire_store(g - 1, pb)
        return carry

    lax.fori_loop(0, (STAGES - NBUF) // NBUF, pair, 0)

    wait_gather(1)
    fire_store(STAGES - 1, 1)
    wait_store(STAGES - 2, 0)
    wait_store(STAGES - 1, 1)


def _sc_gather(idx_flat, table):
    mesh = plsc.VectorSubcoreMesh(core_axis_name="c", subcore_axis_name="s",
                                  num_cores=NC, num_subcores=NS)
    fn = pl.kernel(
        _gather_body,
        out_type=jax.ShapeDtypeStruct((TOTAL, D), jnp.float32),
        mesh=mesh,
        scratch_types=[
            pltpu.VMEM((PER_TILE,), jnp.int32),
            pltpu.VMEM((NIDX, D), jnp.float32),
            pltpu.VMEM((NIDX, D), jnp.float32),
            pltpu.SemaphoreType.DMA,
            pltpu.SemaphoreType.DMA,
            pltpu.SemaphoreType.DMA,
            pltpu.SemaphoreType.DMA,
        ],
        compiler_params=pltpu.CompilerParams(use_tc_tiling_on_sc=False),
    )
    return fn(idx_flat, table)


LBLK_I = 2


def _item_body(x_ref, item_ref):
    for j in range(LBLK_I):
        v = x_ref[j]
        lt = jnp.transpose(v[:, :D])
        rt = jnp.transpose(v[:, D:])
        item_ref[j] = jnp.concatenate([lt, rt], axis=-1)


def _item_finish(x2):
    return pl.pallas_call(
        _item_body,
        grid=(L // LBLK_I,),
        in_specs=[pl.BlockSpec((LBLK_I, B // 2, 128), lambda i: (i, 0, 0))],
        out_specs=pl.BlockSpec((LBLK_I, D, B), lambda i: (i, 0, 0)),
        out_shape=jax.ShapeDtypeStruct((L, D, B), jnp.float32),
    )(x2)


LBLK_P = 8
BBLK = 2048


def _price_body(pt_ref, w_ref, b_ref, o_ref):
    p = pt_ref[...]
    w = w_ref[...].reshape(D, 1)
    bb = b_ref[...].reshape(D, 1)
    o_ref[...] = p[:, None, :] * w[None, :, :] + bb[None, :, :]


def _price_ldb(priceT, W, b):
    return pl.pallas_call(
        _price_body,
        grid=(L // LBLK_P, B // BBLK),
        in_specs=[
            pl.BlockSpec((LBLK_P, BBLK), lambda i, j: (i, j)),
            pl.BlockSpec((1, D), lambda i, j: (0, 0)),
            pl.BlockSpec((1, D), lambda i, j: (0, 0)),
        ],
        out_specs=pl.BlockSpec((LBLK_P, D, BBLK), lambda i, j: (i, 0, j)),
        out_shape=jax.ShapeDtypeStruct((L, D, B), jnp.float32),
    )(priceT, W, b)


_PERM = np.arange(B)
_PERM = (_PERM // 2) + (_PERM % 2) * (B // 2)


@jax.jit
def kernel(item_id, price, emb_table, W, b):
    idx_perm = jnp.take(item_id.T, jnp.asarray(_PERM), axis=1)
    idx_flat = idx_perm.reshape(TOTAL)
    packed = _sc_gather(idx_flat, emb_table)
    x2 = packed.reshape(L, B // 2, 128)
    item_ldb = _item_finish(x2)
    price_ldb = _price_ldb(price.T, W, b.reshape(1, D))
    item_emb = item_ldb.transpose(2, 0, 1)
    price_emb = price_ldb.transpose(2, 0, 1)
    return (item_emb, price_emb)


# speedup vs baseline: 1.7900x; 1.0031x over previous
import functools

import numpy as np
import jax
import jax.numpy as jnp
from jax import lax
from jax.experimental import pallas as pl
from jax.experimental.pallas import tpu as pltpu
from jax.experimental.pallas import tpu_sc as plsc

B = 4096
L = 200
D = 64
NC = 2
NS = 16
NW = NC * NS
TOTAL = B * L
PER_TILE = TOTAL // NW
NIDX = 256
STAGES = PER_TILE // NIDX
NBUF = 2
LANES = 128


def _gather_body(idx_hbm, table_hbm, out_hbm, idx_all, rows0, rows1,
                 gsem0, gsem1, ssem0, ssem1):
    c = lax.axis_index("c")
    s = lax.axis_index("s")
    wid = s * NC + c
    base = wid * PER_TILE

    rows = (rows0, rows1)
    gsem = (gsem0, gsem1)
    ssem = (ssem0, ssem1)

    pltpu.sync_copy(idx_hbm.at[pl.ds(base, PER_TILE)], idx_all)

    def fire_gather(g, b):
        pltpu.async_copy(table_hbm.at[idx_all.at[pl.ds(g * NIDX, NIDX)]],
                         rows[b], gsem[b])

    def wait_gather(b):
        pltpu.make_async_copy(table_hbm.at[idx_all.at[pl.ds(0, NIDX)]],
                              rows[b], gsem[b]).wait()

    def fire_store(g, b):
        pltpu.async_copy(rows[b].at[:, pl.ds(0, D)],
                         out_hbm.at[pl.ds(base + g * NIDX, NIDX)],
                         ssem[b])

    def wait_store(g, b):
        pltpu.make_async_copy(rows[b].at[:, pl.ds(0, D)],
                              out_hbm.at[pl.ds(base + g * NIDX, NIDX)],
                              ssem[b]).wait()

    fire_gather(0, 0)
    fire_gather(1, 1)
    wait_gather(0)
    fire_store(0, 0)

    def pair(k, carry):
        g2 = 2 + 2 * k
        for b in range(NBUF):
            g = g2 + b
            pb = 1 - b
            wait_store(g - 2, b)
            fire_gather(g, b)
            wait_gather(pb)
            fire_store(g - 1, pb)
        return carry

    lax.fori_loop(0, (STAGES - NBUF) // NBUF, pair, 0)

    wait_gather(1)
    fire_store(STAGES - 1, 1)
    wait_store(STAGES - 2, 0)
    wait_store(STAGES - 1, 1)


def _sc_gather(idx_flat, table):
    mesh = plsc.VectorSubcoreMesh(core_axis_name="c", subcore_axis_name="s",
                                  num_cores=NC, num_subcores=NS)
    fn = pl.kernel(
        _gather_body,
        out_type=jax.ShapeDtypeStruct((TOTAL, D), jnp.float32),
        mesh=mesh,
        scratch_types=[
            pltpu.VMEM((PER_TILE,), jnp.int32),
            pltpu.VMEM((NIDX, LANES), jnp.float32),
            pltpu.VMEM((NIDX, LANES), jnp.float32),
            pltpu.SemaphoreType.DMA,
            pltpu.SemaphoreType.DMA,
            pltpu.SemaphoreType.DMA,
            pltpu.SemaphoreType.DMA,
        ],
        compiler_params=pltpu.CompilerParams(use_tc_tiling_on_sc=False),
    )
    return fn(idx_flat, table)


LBLK_I = 2


def _item_body(x_ref, item_ref):
    for j in range(LBLK_I):
        v = x_ref[j]
        lt = jnp.transpose(v[:, :D])
        rt = jnp.transpose(v[:, D:])
        item_ref[j] = jnp.concatenate([lt, rt], axis=-1)


def _item_finish(x2):
    return pl.pallas_call(
        _item_body,
        grid=(L // LBLK_I,),
        in_specs=[pl.BlockSpec((LBLK_I, B // 2, 128), lambda i: (i, 0, 0))],
        out_specs=pl.BlockSpec((LBLK_I, D, B), lambda i: (i, 0, 0)),
        out_shape=jax.ShapeDtypeStruct((L, D, B), jnp.float32),
    )(x2)


LBLK_P = 8
BBLK = 2048


def _price_body(pt_ref, w_ref, b_ref, o_ref):
    p = pt_ref[...]
    w = w_ref[...].reshape(D, 1)
    bb = b_ref[...].reshape(D, 1)
    o_ref[...] = p[:, None, :] * w[None, :, :] + bb[None, :, :]


def _price_ldb(priceT, W, b):
    return pl.pallas_call(
        _price_body,
        grid=(L // LBLK_P, B // BBLK),
        in_specs=[
            pl.BlockSpec((LBLK_P, BBLK), lambda i, j: (i, j)),
            pl.BlockSpec((1, D), lambda i, j: (0, 0)),
            pl.BlockSpec((1, D), lambda i, j: (0, 0)),
        ],
        out_specs=pl.BlockSpec((LBLK_P, D, BBLK), lambda i, j: (i, 0, j)),
        out_shape=jax.ShapeDtypeStruct((L, D, B), jnp.float32),
    )(priceT, W, b)


_PERM = np.arange(B)
_PERM = (_PERM // 2) + (_PERM % 2) * (B // 2)


@jax.jit
def kernel(item_id, price, emb_table, W, b):
    idx_perm = jnp.take(item_id.T, jnp.asarray(_PERM), axis=1)
    idx_flat = idx_perm.reshape(TOTAL)
    table_pad = jnp.pad(emb_table, ((0, 0), (0, LANES - D)))
    packed = _sc_gather(idx_flat, table_pad)
    x2 = packed.reshape(L, B // 2, 128)
    item_ldb = _item_finish(x2)
    price_ldb = _price_ldb(price.T, W, b.reshape(1, D))
    item_emb = item_ldb.transpose(2, 0, 1)
    price_emb = price_ldb.transpose(2, 0, 1)
    return (item_emb, price_emb)


# speedup vs baseline: 1.8383x; 1.0269x over previous
import functools

import numpy as np
import jax
import jax.numpy as jnp
from jax import lax
from jax.experimental import pallas as pl
from jax.experimental.pallas import tpu as pltpu
from jax.experimental.pallas import tpu_sc as plsc

B = 4096
L = 200
D = 64
NC = 2
NS = 16
NW = NC * NS
TOTAL = B * L
PER_TILE = TOTAL // NW
NIDX = 320
STAGES = PER_TILE // NIDX
NBUF = 2
LANES = 128


def _gather_body(idx_hbm, table_hbm, out_hbm, idx_all, rows0, rows1,
                 gsem0, gsem1, ssem0, ssem1):
    c = lax.axis_index("c")
    s = lax.axis_index("s")
    wid = s * NC + c
    base = wid * PER_TILE

    rows = (rows0, rows1)
    gsem = (gsem0, gsem1)
    ssem = (ssem0, ssem1)

    pltpu.sync_copy(idx_hbm.at[pl.ds(base, PER_TILE)], idx_all)

    def fire_gather(g, b):
        pltpu.async_copy(table_hbm.at[idx_all.at[pl.ds(g * NIDX, NIDX)]],
                         rows[b], gsem[b])

    def wait_gather(b):
        pltpu.make_async_copy(table_hbm.at[idx_all.at[pl.ds(0, NIDX)]],
                              rows[b], gsem[b]).wait()

    def fire_store(g, b):
        pltpu.async_copy(rows[b].at[:, pl.ds(0, D)],
                         out_hbm.at[pl.ds(base + g * NIDX, NIDX)],
                         ssem[b])

    def wait_store(g, b):
        pltpu.make_async_copy(rows[b].at[:, pl.ds(0, D)],
                              out_hbm.at[pl.ds(base + g * NIDX, NIDX)],
                              ssem[b]).wait()

    fire_gather(0, 0)
    fire_gather(1, 1)
    wait_gather(0)
    fire_store(0, 0)

    def pair(k, carry):
        g2 = 2 + 2 * k
        for b in range(NBUF):
            g = g2 + b
            pb = 1 - b
            wait_store(g - 2, b)
            fire_gather(g, b)
            wait_gather(pb)
            fire_store(g - 1, pb)
        return carry

    lax.fori_loop(0, (STAGES - NBUF) // NBUF, pair, 0)

    wait_gather(1)
    fire_store(STAGES - 1, 1)
    wait_store(STAGES - 2, 0)
    wait_store(STAGES - 1, 1)


def _sc_gather(idx_flat, table):
    mesh = plsc.VectorSubcoreMesh(core_axis_name="c", subcore_axis_name="s",
                                  num_cores=NC, num_subcores=NS)
    fn = pl.kernel(
        _gather_body,
        out_type=jax.ShapeDtypeStruct((TOTAL, D), jnp.float32),
        mesh=mesh,
        scratch_types=[
            pltpu.VMEM((PER_TILE,), jnp.int32),
            pltpu.VMEM((NIDX, LANES), jnp.float32),
            pltpu.VMEM((NIDX, LANES), jnp.float32),
            pltpu.SemaphoreType.DMA,
            pltpu.SemaphoreType.DMA,
            pltpu.SemaphoreType.DMA,
            pltpu.SemaphoreType.DMA,
        ],
        compiler_params=pltpu.CompilerParams(use_tc_tiling_on_sc=False),
    )
    return fn(idx_flat, table)


LBLK_I = 4


def _item_body(x_ref, item_ref):
    for j in range(LBLK_I):
        v = x_ref[j]
        lt = jnp.transpose(v[:, :D])
        rt = jnp.transpose(v[:, D:])
        item_ref[j] = jnp.concatenate([lt, rt], axis=-1)


def _item_finish(x2):
    return pl.pallas_call(
        _item_body,
        grid=(L // LBLK_I,),
        in_specs=[pl.BlockSpec((LBLK_I, B // 2, 128), lambda i: (i, 0, 0))],
        out_specs=pl.BlockSpec((LBLK_I, D, B), lambda i: (i, 0, 0)),
        out_shape=jax.ShapeDtypeStruct((L, D, B), jnp.float32),
    )(x2)


LBLK_P = 8
BBLK = 2048


def _price_body(pt_ref, w_ref, b_ref, o_ref):
    p = pt_ref[...]
    w = w_ref[...].reshape(D, 1)
    bb = b_ref[...].reshape(D, 1)
    o_ref[...] = p[:, None, :] * w[None, :, :] + bb[None, :, :]


def _price_ldb(priceT, W, b):
    return pl.pallas_call(
        _price_body,
        grid=(L // LBLK_P, B // BBLK),
        in_specs=[
            pl.BlockSpec((LBLK_P, BBLK), lambda i, j: (i, j)),
            pl.BlockSpec((1, D), lambda i, j: (0, 0)),
            pl.BlockSpec((1, D), lambda i, j: (0, 0)),
        ],
        out_specs=pl.BlockSpec((LBLK_P, D, BBLK), lambda i, j: (i, 0, j)),
        out_shape=jax.ShapeDtypeStruct((L, D, B), jnp.float32),
    )(priceT, W, b)


_PERM = np.arange(B)
_PERM = (_PERM // 2) + (_PERM % 2) * (B // 2)


@jax.jit
def kernel(item_id, price, emb_table, W, b):
    idx_perm = jnp.take(item_id.T, jnp.asarray(_PERM), axis=1)
    idx_flat = idx_perm.reshape(TOTAL)
    table_pad = jnp.pad(emb_table, ((0, 0), (0, LANES - D)))
    packed = _sc_gather(idx_flat, table_pad)
    x2 = packed.reshape(L, B // 2, 128)
    item_ldb = _item_finish(x2)
    price_ldb = _price_ldb(price.T, W, b.reshape(1, D))
    item_emb = item_ldb.transpose(2, 0, 1)
    price_emb = price_ldb.transpose(2, 0, 1)
    return (item_emb, price_emb)
